# baseline (speedup 1.0000x reference)
import functools

import jax
import jax.numpy as jnp
from jax import lax
from jax.experimental import pallas as pl
from jax.experimental.pallas import tpu as pltpu
from jax.experimental.pallas import tpu_sc as plsc

BATCH = 16384
EMBED = 32
HID = 64
ACT = 6

NUM_CORES = 2
NUM_SUBCORES = 16
NUM_WORKERS = NUM_CORES * NUM_SUBCORES
B_PER_W = BATCH // NUM_WORKERS


def _sc_gather(table, idx):
    mesh = plsc.VectorSubcoreMesh(core_axis_name="c", subcore_axis_name="s")

    @functools.partial(
        pl.kernel,
        mesh=mesh,
        out_type=jax.ShapeDtypeStruct((BATCH, EMBED), jnp.float32),
        scratch_types=[
            pltpu.VMEM((B_PER_W + 16,), jnp.int32),
            pltpu.SemaphoreType.DMA,
            pltpu.SemaphoreType.DMA,
        ],
    )
    def gather_kernel(idx_hbm, table_hbm, out_hbm, idx_v, sem_i, sem_g):
        wid = lax.axis_index("s") * NUM_CORES + lax.axis_index("c")
        base = wid * B_PER_W
        pltpu.async_copy(
            idx_hbm.at[pl.ds(base, B_PER_W)], idx_v.at[pl.ds(0, B_PER_W)], sem_i
        ).wait()

        @pl.loop(0, B_PER_W)
        def _(i):
            r = idx_v[pl.ds(i, 16)][0]
            pltpu.async_copy(
                table_hbm.at[pl.ds(r, 1)], out_hbm.at[pl.ds(base + i, 1)], sem_g
            )

        pltpu.make_async_copy(
            table_hbm.at[pl.ds(0, B_PER_W)],
            out_hbm.at[pl.ds(base, B_PER_W)],
            sem_g,
        ).wait()

    return gather_kernel(idx, table)


def _mlp_body(x_ref, w1_ref, b1_ref, w2_ref, b2_ref, o_ref):
    h = jnp.dot(x_ref[...], w1_ref[...], preferred_element_type=jnp.float32)
    h = jnp.maximum(h + b1_ref[...], 0.0)
    o = jnp.dot(h, w2_ref[...], preferred_element_type=jnp.float32)
    o_ref[...] = o + b2_ref[...]


def _tc_mlp(x, W1, b1, W2, b2):
    nblk = 8
    blk = BATCH // nblk
    return pl.pallas_call(
        _mlp_body,
        grid=(nblk,),
        in_specs=[
            pl.BlockSpec((blk, EMBED), lambda i: (i, 0)),
            pl.BlockSpec((EMBED, HID), lambda i: (0, 0)),
            pl.BlockSpec((1, HID), lambda i: (0, 0)),
            pl.BlockSpec((HID, ACT), lambda i: (0, 0)),
            pl.BlockSpec((1, ACT), lambda i: (0, 0)),
        ],
        out_specs=pl.BlockSpec((blk, ACT), lambda i: (i, 0)),
        out_shape=jax.ShapeDtypeStruct((BATCH, ACT), jnp.float32),
    )(x, W1, b1.reshape(1, HID), W2, b2.reshape(1, ACT))


def kernel(state, table, W1, b1, W2, b2):
    x = _sc_gather(table, state.astype(jnp.int32))
    return _tc_mlp(x, W1, b1, W2, b2)


# speedup vs baseline: 1.0256x; 1.0256x over previous
import functools

import jax
import jax.numpy as jnp
from jax import lax
from jax.experimental import pallas as pl
from jax.experimental.pallas import tpu as pltpu
from jax.experimental.pallas import tpu_sc as plsc

BATCH = 16384
EMBED = 32
HID = 64
ACT = 6
PACK = 4
ROW = EMBED * PACK
NROWS = 1000000
NPACKED = NROWS // PACK

NUM_CORES = 2
NUM_SUBCORES = 16
NUM_WORKERS = NUM_CORES * NUM_SUBCORES
B_PER_W = BATCH // NUM_WORKERS

BLK_D = 80
BLK_S = BLK_D * PACK
NBLK = NPACKED // BLK_D


def _sc_repack(table):
    mesh = plsc.VectorSubcoreMesh(core_axis_name="c", subcore_axis_name="s")

    @functools.partial(
        pl.kernel,
        mesh=mesh,
        out_type=jax.ShapeDtypeStruct((NPACKED, ROW), jnp.float32),
        scratch_types=[
            pltpu.VMEM((BLK_S, EMBED), jnp.float32),
            pltpu.VMEM((BLK_S, EMBED), jnp.float32),
            pltpu.VMEM((BLK_D, ROW), jnp.float32),
            pltpu.VMEM((BLK_D, ROW), jnp.float32),
            pltpu.SemaphoreType.DMA,
            pltpu.SemaphoreType.DMA,
            pltpu.SemaphoreType.DMA,
            pltpu.SemaphoreType.DMA,
        ],
    )
    def repack_kernel(table_hbm, out_hbm, in0, in1, out0, out1,
                      si0, si1, so0, so1):
        wid = lax.axis_index("s") * NUM_CORES + lax.axis_index("c")
        n_blk = jnp.where(wid < 21, 98, 97)
        base_blk = wid * 97 + jnp.minimum(wid, 21)

        def repack(in_v, out_v):
            for d in range(BLK_D):
                for j in range(PACK):
                    for k in range(2):
                        src = (pl.ds(PACK * d + j, 1), pl.ds(16 * k, 16))
                        dst = (pl.ds(d, 1), pl.ds(EMBED * j + 16 * k, 16))
                        out_v.at[*dst][...] = in_v.at[*src][...]

        def fetch(b, in_v, sem):
            pltpu.async_copy(
                table_hbm.at[pl.ds((base_blk + b) * BLK_S, BLK_S)], in_v, sem
            )

        def put(b, out_v, sem):
            pltpu.async_copy(
                out_v, out_hbm.at[pl.ds((base_blk + b) * BLK_D, BLK_D)], sem
            )

        fetch(0, in0, si0)

        @pl.loop(0, n_blk, step=2)
        def _(b):
            @pl.when(b + 1 < n_blk)
            def _():
                fetch(b + 1, in1, si1)
            pltpu.make_async_copy(table_hbm.at[pl.ds(0, BLK_S)], in0, si0).wait()
            @pl.when(b >= 2)
            def _():
                pltpu.make_async_copy(out0, out_hbm.at[pl.ds(0, BLK_D)], so0).wait()
            repack(in0, out0)
            put(b, out0, so0)

            @pl.when(b + 2 < n_blk)
            def _():
                fetch(b + 2, in0, si0)

            @pl.when(b + 1 < n_blk)
            def _():
                pltpu.make_async_copy(table_hbm.at[pl.ds(0, BLK_S)], in1, si1).wait()
                @pl.when(b >= 2)
                def _():
                    pltpu.make_async_copy(out1, out_hbm.at[pl.ds(0, BLK_D)], so1).wait()
                repack(in1, out1)
                put(b + 1, out1, so1)

        @pl.when(n_blk >= 2)
        def _():
            pltpu.make_async_copy(out1, out_hbm.at[pl.ds(0, BLK_D)], so1).wait()
        pltpu.make_async_copy(out0, out_hbm.at[pl.ds(0, BLK_D)], so0).wait()

    return repack_kernel(table)


def _sc_gather(table128, idx_hi):
    mesh = plsc.VectorSubcoreMesh(core_axis_name="c", subcore_axis_name="s")

    @functools.partial(
        pl.kernel,
        mesh=mesh,
        out_type=jax.ShapeDtypeStruct((BATCH, ROW), jnp.float32),
        scratch_types=[
            pltpu.VMEM((B_PER_W,), jnp.int32),
            pltpu.VMEM((B_PER_W, ROW), jnp.float32),
            pltpu.SemaphoreType.DMA,
        ],
    )
    def gather_kernel(idx_hbm, table_hbm, out_hbm, idx_v, rows_v, sem):
        wid = lax.axis_index("s") * NUM_CORES + lax.axis_index("c")
        base = wid * B_PER_W
        pltpu.sync_copy(idx_hbm.at[pl.ds(base, B_PER_W)], idx_v)
        pltpu.async_copy(table_hbm.at[idx_v], rows_v, sem).wait()
        pltpu.sync_copy(rows_v, out_hbm.at[pl.ds(base, B_PER_W)])

    return gather_kernel(idx_hi, table128)


def _mlp_body(x_ref, k_ref, w1_ref, b1_ref, w2_ref, b2_ref, o_ref):
    h = jnp.dot(x_ref[...], w1_ref[...], preferred_element_type=jnp.float32)
    h = jnp.maximum(h + b1_ref[...], 0.0)
    q4 = jnp.dot(h, w2_ref[...], preferred_element_type=jnp.float32)
    q4 = q4 + b2_ref[...]
    group = lax.broadcasted_iota(jnp.int32, q4.shape, 1) // ACT
    q4 = jnp.where(group == k_ref[...], q4, 0.0)
    o_ref[...] = (q4[:, 0:ACT] + q4[:, ACT:2 * ACT]
                  + q4[:, 2 * ACT:3 * ACT] + q4[:, 3 * ACT:4 * ACT])


def _tc_mlp(x, k, W1s, b1s, W2s, b2s):
    nblk = 8
    blk = BATCH // nblk
    return pl.pallas_call(
        _mlp_body,
        grid=(nblk,),
        in_specs=[
            pl.BlockSpec((blk, ROW), lambda i: (i, 0)),
            pl.BlockSpec((blk, 1), lambda i: (i, 0)),
            pl.BlockSpec((ROW, PACK * HID), lambda i: (0, 0)),
            pl.BlockSpec((1, PACK * HID), lambda i: (0, 0)),
            pl.BlockSpec((PACK * HID, PACK * ACT), lambda i: (0, 0)),
            pl.BlockSpec((1, PACK * ACT), lambda i: (0, 0)),
        ],
        out_specs=pl.BlockSpec((blk, ACT), lambda i: (i, 0)),
        out_shape=jax.ShapeDtypeStruct((BATCH, ACT), jnp.float32),
    )(x, k, W1s, b1s, W2s, b2s)


def kernel(state, table, W1, b1, W2, b2):
    state = state.astype(jnp.int32)
    table128 = _sc_repack(table)
    x = _sc_gather(table128, state // PACK)
    k = (state % PACK).reshape(BATCH, 1)
    W1s = jax.scipy.linalg.block_diag(W1, W1, W1, W1)
    W2s = jax.scipy.linalg.block_diag(W2, W2, W2, W2)
    b1s = jnp.tile(b1, PACK).reshape(1, PACK * HID)
    b2s = jnp.tile(b2, PACK).reshape(1, PACK * ACT)
    return _tc_mlp(x, k, W1s, b1s, W2s, b2s)


# speedup vs baseline: 1.7638x; 1.7197x over previous
import functools

import jax
import jax.numpy as jnp
from jax import lax
from jax.experimental import pallas as pl
from jax.experimental.pallas import tpu as pltpu
from jax.experimental.pallas import tpu_sc as plsc

BATCH = 16384
EMBED = 32
HID = 64
ACT = 6

NUM_CORES = 2
NUM_SUBCORES = 16
NUM_WORKERS = NUM_CORES * NUM_SUBCORES
B_PER_W = BATCH // NUM_WORKERS


def _sc_gather(table, idx):
    mesh = plsc.VectorSubcoreMesh(core_axis_name="c", subcore_axis_name="s")

    @functools.partial(
        pl.kernel,
        mesh=mesh,
        out_type=jax.ShapeDtypeStruct((BATCH, EMBED), jnp.float32),
        scratch_types=[
            pltpu.VMEM((B_PER_W,), jnp.int32),
            pltpu.VMEM((B_PER_W, EMBED), jnp.float32),
            pltpu.SemaphoreType.DMA,
            pltpu.SemaphoreType.DMA,
            pltpu.SemaphoreType.DMA,
            pltpu.SemaphoreType.DMA,
            pltpu.SemaphoreType.DMA,
            pltpu.SemaphoreType.DMA,
            pltpu.SemaphoreType.DMA,
            pltpu.SemaphoreType.DMA,
            pltpu.SemaphoreType.DMA,
        ],
        compiler_params=pltpu.CompilerParams(use_tc_tiling_on_sc=True),
    )
    def gather_kernel(idx_hbm, table_hbm, out_hbm, idx_v, rows_v, sem_i,
                      s0, s1, s2, s3, s4, s5, s6, s7):
        sems = (s0, s1, s2, s3, s4, s5, s6, s7)
        wid = lax.axis_index("s") * NUM_CORES + lax.axis_index("c")
        base = wid * B_PER_W
        pltpu.async_copy(idx_hbm.at[pl.ds(base, B_PER_W)], idx_v, sem_i).wait()

        @pl.loop(0, B_PER_W, step=16)
        def _(i):
            vec = idx_v[pl.ds(i, 16)]
            for j in range(16):
                pltpu.async_copy(
                    table_hbm.at[pl.ds(vec[j], 1)],
                    rows_v.at[pl.ds(i + j, 1)],
                    sems[j % 8],
                )

        for q in range(8):
            pltpu.make_async_copy(
                table_hbm.at[pl.ds(0, B_PER_W // 8)],
                rows_v.at[pl.ds(q * (B_PER_W // 8), B_PER_W // 8)],
                sems[q],
            ).wait()
        pltpu.async_copy(rows_v, out_hbm.at[pl.ds(base, B_PER_W)], sem_i).wait()

    return gather_kernel(idx, table)


def _mlp_body(x_ref, w1_ref, b1_ref, w2_ref, b2_ref, o_ref):
    h = jnp.dot(x_ref[...], w1_ref[...], preferred_element_type=jnp.float32)
    h = jnp.maximum(h + b1_ref[...], 0.0)
    o = jnp.dot(h, w2_ref[...], preferred_element_type=jnp.float32)
    o_ref[...] = o + b2_ref[...]


def _tc_mlp(x, W1, b1, W2, b2):
    nblk = 8
    blk = BATCH // nblk
    return pl.pallas_call(
        _mlp_body,
        grid=(nblk,),
        in_specs=[
            pl.BlockSpec((blk, EMBED), lambda i: (i, 0)),
            pl.BlockSpec((EMBED, HID), lambda i: (0, 0)),
            pl.BlockSpec((1, HID), lambda i: (0, 0)),
            pl.BlockSpec((HID, ACT), lambda i: (0, 0)),
            pl.BlockSpec((1, ACT), lambda i: (0, 0)),
        ],
        out_specs=pl.BlockSpec((blk, ACT), lambda i: (i, 0)),
        out_shape=jax.ShapeDtypeStruct((BATCH, ACT), jnp.float32),
    )(x, W1, b1.reshape(1, HID), W2, b2.reshape(1, ACT))


def kernel(state, table, W1, b1, W2, b2):
    x = _sc_gather(table, state.astype(jnp.int32))
    return _tc_mlp(x, W1, b1, W2, b2)
